# baseline (speedup 1.0000x reference)
import jax
import jax.numpy as jnp
from jax.experimental import pallas as pl
from jax.experimental.pallas import tpu as pltpu

_CHUNK = 5000
_NBUF = 5


def _mlp_gate_pipeline(x_hbm, pe_v, w1_v, b1_v, w2_v, b2_v, o_hbm,
                       in_buf, out_buf, in_sems, out_sems):
    n = x_hbm.shape[0]
    nchunks = n // _CHUNK

    def in_copy(i, slot):
        return pltpu.make_async_copy(
            x_hbm.at[pl.ds(i * _CHUNK, _CHUNK), :], in_buf.at[slot],
            in_sems.at[slot])

    def out_copy(i, slot):
        return pltpu.make_async_copy(
            out_buf.at[slot], o_hbm.at[pl.ds(i * _CHUNK, _CHUNK), :],
            out_sems.at[slot])

    for s in range(min(_NBUF, nchunks)):
        in_copy(s, s).start()

    for i in range(nchunks):
        slot = i % _NBUF
        in_copy(i, slot).wait()
        x = in_buf[slot] + pe_v[...]
        h = jnp.dot(x, w1_v[...], preferred_element_type=jnp.float32)
        h = jnp.maximum(h + b1_v[...], 0.0)
        m = jnp.dot(h, w2_v[...], preferred_element_type=jnp.float32)
        m = jnp.maximum(m + b2_v[...], 0.0)
        if i >= _NBUF:
            out_copy(i - _NBUF, slot).wait()
        out_buf[slot] = x * jax.nn.sigmoid(m)
        out_copy(i, slot).start()
        if i + _NBUF < nchunks:
            in_copy(i + _NBUF, slot).start()

    for i in range(max(0, nchunks - _NBUF), nchunks):
        out_copy(i, i % _NBUF).wait()


def kernel(graph, input, W1, b1, W2, b2, pe_coff, pe, t):
    n, d = input.shape
    pe_row = pe_coff * jax.lax.dynamic_index_in_dim(pe, t + 1, axis=0, keepdims=True)
    w2_rep = jnp.broadcast_to(W2[:, 1:2], (d, d))
    b2_col = b2[1].reshape(1, 1)
    b1_row = b1.reshape(1, d)
    assert n % _CHUNK == 0

    vmem = pl.BlockSpec(memory_space=pltpu.VMEM)
    return pl.pallas_call(
        _mlp_gate_pipeline,
        in_specs=[
            pl.BlockSpec(memory_space=pl.ANY),
            vmem, vmem, vmem, vmem, vmem,
        ],
        out_specs=pl.BlockSpec(memory_space=pl.ANY),
        out_shape=jax.ShapeDtypeStruct((n, d), jnp.float32),
        scratch_shapes=[
            pltpu.VMEM((_NBUF, _CHUNK, d), jnp.float32),
            pltpu.VMEM((_NBUF, _CHUNK, d), jnp.float32),
            pltpu.SemaphoreType.DMA((_NBUF,)),
            pltpu.SemaphoreType.DMA((_NBUF,)),
        ],
    )(input, pe_row, W1, b1_row, w2_rep, b2_col)


# speedup vs baseline: 1.2704x; 1.2704x over previous
import jax
import jax.numpy as jnp
from jax.experimental import pallas as pl
from jax.experimental.pallas import tpu as pltpu


def _mlp_gate_block(x_ref, pe_ref, w1_ref, b1_ref, w2_ref, b2_ref, o_ref):
    x = x_ref[...] + pe_ref[...]
    h = jnp.dot(x, w1_ref[...], preferred_element_type=jnp.float32)
    h = jnp.maximum(h + b1_ref[...], 0.0)
    m = jnp.dot(h, w2_ref[...], preferred_element_type=jnp.float32)
    m = jnp.maximum(m + b2_ref[...], 0.0)
    o_ref[...] = x * jax.nn.sigmoid(m)


def kernel(graph, input, W1, b1, W2, b2, pe_coff, pe, t):
    n, d = input.shape
    pe_row = pe_coff * jax.lax.dynamic_index_in_dim(pe, t + 1, axis=0, keepdims=True)
    w1_b = W1
    w2_col = W2[:, 1:2]
    b2_col = b2[1].reshape(1, 1)
    b1_row = b1.reshape(1, d)

    block = 20000
    assert n % block == 0
    grid = (n // block,)

    return pl.pallas_call(
        _mlp_gate_block,
        grid=grid,
        in_specs=[
            pl.BlockSpec((block, d), lambda i: (i, 0)),
            pl.BlockSpec((1, d), lambda i: (0, 0)),
            pl.BlockSpec((d, d), lambda i: (0, 0)),
            pl.BlockSpec((1, d), lambda i: (0, 0)),
            pl.BlockSpec((d, 1), lambda i: (0, 0)),
            pl.BlockSpec((1, 1), lambda i: (0, 0)),
        ],
        out_specs=pl.BlockSpec((block, d), lambda i: (i, 0)),
        out_shape=jax.ShapeDtypeStruct((n, d), jnp.float32),
        compiler_params=pltpu.CompilerParams(
            dimension_semantics=("parallel",),
        ),
    )(input, pe_row, w1_b, b1_row, w2_col, b2_col)


# speedup vs baseline: 1.2834x; 1.0102x over previous
import jax
import jax.numpy as jnp
from jax.experimental import pallas as pl
from jax.experimental.pallas import tpu as pltpu


def _mlp_gate_block(x_ref, pe_ref, w1_ref, b1_ref, w2_ref, b2_ref, o_ref):
    x = x_ref[...] + pe_ref[...]
    h = jnp.dot(x, w1_ref[...], preferred_element_type=jnp.float32)
    h = jnp.maximum(h + b1_ref[...], 0.0)
    m = jnp.dot(h, w2_ref[...], preferred_element_type=jnp.float32)
    m = jnp.maximum(m + b2_ref[...], 0.0)
    o_ref[...] = x * jax.nn.sigmoid(m)


def kernel(graph, input, W1, b1, W2, b2, pe_coff, pe, t):
    n, d = input.shape
    pe_row = pe_coff * jax.lax.dynamic_index_in_dim(pe, t + 1, axis=0, keepdims=True)
    w1_b = W1
    w2_col = jnp.broadcast_to(W2[:, 1:2], (d, d))
    b2_col = b2[1].reshape(1, 1)
    b1_row = b1.reshape(1, d)

    block = 20000
    assert n % block == 0
    grid = (n // block,)

    return pl.pallas_call(
        _mlp_gate_block,
        grid=grid,
        in_specs=[
            pl.BlockSpec((block, d), lambda i: (i, 0)),
            pl.BlockSpec((1, d), lambda i: (0, 0)),
            pl.BlockSpec((d, d), lambda i: (0, 0)),
            pl.BlockSpec((1, d), lambda i: (0, 0)),
            pl.BlockSpec((d, d), lambda i: (0, 0)),
            pl.BlockSpec((1, 1), lambda i: (0, 0)),
        ],
        out_specs=pl.BlockSpec((block, d), lambda i: (i, 0)),
        out_shape=jax.ShapeDtypeStruct((n, d), jnp.float32),
        compiler_params=pltpu.CompilerParams(
            dimension_semantics=("parallel",),
        ),
    )(input, pe_row, w1_b, b1_row, w2_col, b2_col)
